# Initial kernel scaffold; baseline (speedup 1.0000x reference)
import functools

import jax
import jax.numpy as jnp
from jax import lax
from jax.experimental import pallas as pl
from jax.experimental.pallas import tpu as pltpu
from jax.experimental.pallas import tpu_sc as plsc

N = 10000
E = 320000
D = 128
H = 128
C = 64
K = 10
ALPHA = 0.1

NC = 2
NS = 16
LANES = 16
EDGES_PER_TILE = E // (NC * NS)
CHUNK = 80
NCHUNK = EDGES_PER_TILE // CHUNK
ROWS_PER_TILE = N // NS

ROW_BLK = 2000


def _mlp_body(x_ref, w1_ref, b1_ref, w2_ref, b2_ref, h_ref, ah_ref):
    h1 = jnp.maximum(
        jnp.dot(x_ref[...], w1_ref[...], preferred_element_type=jnp.float32)
        + b1_ref[...], 0.0)
    h2 = (jnp.dot(h1, w2_ref[...], preferred_element_type=jnp.float32)
          + b2_ref[...])
    h_ref[...] = h2
    ah_ref[...] = ALPHA * h2


def _mlp(features, W1, b1, W2, b2):
    grid = (N // ROW_BLK,)
    return pl.pallas_call(
        _mlp_body,
        grid=grid,
        in_specs=[
            pl.BlockSpec((ROW_BLK, D), lambda i: (i, 0)),
            pl.BlockSpec((D, H), lambda i: (0, 0)),
            pl.BlockSpec((1, H), lambda i: (0, 0)),
            pl.BlockSpec((H, C), lambda i: (0, 0)),
            pl.BlockSpec((1, C), lambda i: (0, 0)),
        ],
        out_specs=[
            pl.BlockSpec((ROW_BLK, C), lambda i: (i, 0)),
            pl.BlockSpec((ROW_BLK, C), lambda i: (i, 0)),
        ],
        out_shape=[
            jax.ShapeDtypeStruct((N, C), jnp.float32),
            jax.ShapeDtypeStruct((N, C), jnp.float32),
        ],
    )(features, W1, b1.reshape(1, H), W2, b2.reshape(1, C))


def _combine_body(p_ref, o_ref):
    o_ref[...] = p_ref[0] + p_ref[1]


def _combine(partials):
    grid = (N // ROW_BLK,)
    return pl.pallas_call(
        _combine_body,
        grid=grid,
        in_specs=[pl.BlockSpec((NC, ROW_BLK, C), lambda i: (0, i, 0))],
        out_specs=pl.BlockSpec((ROW_BLK, C), lambda i: (i, 0)),
        out_shape=jax.ShapeDtypeStruct((N, C), jnp.float32),
    )(partials)


def _prop_body(h_hbm, src_hbm, dst_hbm, w_hbm, init_hbm, out_hbm,
               src_v, dst_v, w_v, rows_v, agg_sh):
    c = lax.axis_index("c")
    s = lax.axis_index("s")

    pltpu.sync_copy(src_hbm.at[c, s], src_v)
    pltpu.sync_copy(dst_hbm.at[c, s], dst_v)
    pltpu.sync_copy(w_hbm.at[c, s], w_v)
    pltpu.sync_copy(init_hbm.at[c].at[pl.ds(s * ROWS_PER_TILE, ROWS_PER_TILE)],
                    agg_sh.at[pl.ds(s * ROWS_PER_TILE, ROWS_PER_TILE)])
    plsc.subcore_barrier()

    @pl.loop(0, NCHUNK)
    def _(j):
        pltpu.sync_copy(h_hbm.at[src_v.at[j]], rows_v)
        for e5 in range(CHUNK // LANES):
            w16 = w_v[j, pl.ds(e5 * LANES, LANES)] * (1.0 - ALPHA)
            for e in range(LANES):
                wb = jnp.take(w16, jnp.full((LANES,), e, jnp.int32),
                              mode="promise_in_bounds")
                row = e5 * LANES + e
                for f in range(C // LANES):
                    sl = (row, pl.ds(f * LANES, LANES))
                    rows_v[sl] = rows_v[sl] * wb
        pltpu.sync_copy(rows_v, agg_sh.at[dst_v.at[j]], add=True)

    plsc.subcore_barrier()
    pltpu.sync_copy(agg_sh.at[pl.ds(s * ROWS_PER_TILE, ROWS_PER_TILE)],
                    out_hbm.at[c].at[pl.ds(s * ROWS_PER_TILE, ROWS_PER_TILE)])


def _make_prop():
    mesh = plsc.VectorSubcoreMesh(core_axis_name="c", subcore_axis_name="s")
    return pl.kernel(
        _prop_body,
        mesh=mesh,
        out_type=jax.ShapeDtypeStruct((NC, N, C), jnp.float32),
        scratch_types=[
            pltpu.VMEM((NCHUNK, CHUNK), jnp.int32),
            pltpu.VMEM((NCHUNK, CHUNK), jnp.int32),
            pltpu.VMEM((NCHUNK, CHUNK), jnp.float32),
            pltpu.VMEM((CHUNK, C), jnp.float32),
            pltpu.VMEM_SHARED((N, C), jnp.float32),
        ],
    )


def kernel(features, edge_weight, edge_index, W1, b1, W2, b2):
    h0, ah0 = _mlp(features, W1, b1, W2, b2)
    src = edge_index[0].reshape(NC, NS, NCHUNK, CHUNK)
    dst = edge_index[1].reshape(NC, NS, NCHUNK, CHUNK)
    w = edge_weight.reshape(NC, NS, NCHUNK, CHUNK)
    init = jnp.stack([ah0, jnp.zeros_like(ah0)])
    prop = _make_prop()
    h = h0
    for _ in range(K):
        partials = prop(h, src, dst, w, init)
        h = _combine(partials)
    return h


# speedup vs baseline: 7.8865x; 7.8865x over previous
import functools

import jax
import jax.numpy as jnp
from jax import lax
from jax.experimental import pallas as pl
from jax.experimental.pallas import tpu as pltpu
from jax.experimental.pallas import tpu_sc as plsc

N = 10000
E = 320000
D = 128
H = 128
C = 64
K = 10
ALPHA = 0.1

NC = 2
NS = 16
LANES = 16
EDGES_PER_TILE = E // (NC * NS)
CHUNK = 80
NCHUNK = EDGES_PER_TILE // CHUNK
NPAD = 10240
ROWS_PER_TILE = NPAD // NS

ROW_BLK = 2000
CMB_BLK = 2048


def _mlp_body(x_ref, w1_ref, b1_ref, w2_ref, b2_ref, h_ref, ah_ref):
    h1 = jnp.maximum(
        jnp.dot(x_ref[...], w1_ref[...], preferred_element_type=jnp.float32)
        + b1_ref[...], 0.0)
    h2 = (jnp.dot(h1, w2_ref[...], preferred_element_type=jnp.float32)
          + b2_ref[...])
    h_ref[...] = h2
    ah_ref[...] = ALPHA * h2


def _mlp(features, W1, b1, W2, b2):
    grid = (N // ROW_BLK,)
    return pl.pallas_call(
        _mlp_body,
        grid=grid,
        in_specs=[
            pl.BlockSpec((ROW_BLK, D), lambda i: (i, 0)),
            pl.BlockSpec((D, H), lambda i: (0, 0)),
            pl.BlockSpec((1, H), lambda i: (0, 0)),
            pl.BlockSpec((H, C), lambda i: (0, 0)),
            pl.BlockSpec((1, C), lambda i: (0, 0)),
        ],
        out_specs=[
            pl.BlockSpec((ROW_BLK, C), lambda i: (i, 0)),
            pl.BlockSpec((ROW_BLK, C), lambda i: (i, 0)),
        ],
        out_shape=[
            jax.ShapeDtypeStruct((N, C), jnp.float32),
            jax.ShapeDtypeStruct((N, C), jnp.float32),
        ],
    )(features, W1, b1.reshape(1, H), W2, b2.reshape(1, C))


def _combine_body(p_ref, o_ref):
    o_ref[...] = p_ref[0] + p_ref[1]


def _combine(partials):
    grid = (NPAD // CMB_BLK,)
    return pl.pallas_call(
        _combine_body,
        grid=grid,
        in_specs=[pl.BlockSpec((NC, CMB_BLK, C), lambda i: (0, i, 0))],
        out_specs=pl.BlockSpec((CMB_BLK, C), lambda i: (i, 0)),
        out_shape=jax.ShapeDtypeStruct((NPAD, C), jnp.float32),
    )(partials)


_GATHER_DNUMS = lax.GatherDimensionNumbers(
    offset_dims=(), collapsed_slice_dims=(0,), start_index_map=(0,))


def _bcast_lane(vec, lane):
    idx = jnp.full((LANES, 1), lane, jnp.int32)
    return lax.gather(vec, idx, _GATHER_DNUMS, slice_sizes=(1,),
                      mode=lax.GatherScatterMode.PROMISE_IN_BOUNDS)


def _prop_body(h_hbm, src_hbm, dst_hbm, w_hbm, init_hbm, out_hbm,
               src_v, dst_v, w_v, rows_v, agg_sh):
    c = lax.axis_index("c")
    s = lax.axis_index("s")

    pltpu.sync_copy(src_hbm.at[c, s], src_v)
    pltpu.sync_copy(dst_hbm.at[c, s], dst_v)
    pltpu.sync_copy(w_hbm.at[c, s], w_v)
    pltpu.sync_copy(init_hbm.at[c].at[pl.ds(s * ROWS_PER_TILE, ROWS_PER_TILE)],
                    agg_sh.at[pl.ds(s * ROWS_PER_TILE, ROWS_PER_TILE)])
    plsc.subcore_barrier()

    @pl.loop(0, NCHUNK)
    def _(j):
        pltpu.sync_copy(h_hbm.at[src_v.at[j]], rows_v)
        for e5 in range(CHUNK // LANES):
            w16 = w_v[j, pl.ds(e5 * LANES, LANES)] * (1.0 - ALPHA)
            for e in range(LANES):
                wb = _bcast_lane(w16, e)
                row = e5 * LANES + e
                for f in range(C // LANES):
                    sl = (row, pl.ds(f * LANES, LANES))
                    rows_v[sl] = rows_v[sl] * wb
        pltpu.sync_copy(rows_v, agg_sh.at[dst_v.at[j]], add=True)

    plsc.subcore_barrier()
    pltpu.sync_copy(agg_sh.at[pl.ds(s * ROWS_PER_TILE, ROWS_PER_TILE)],
                    out_hbm.at[c].at[pl.ds(s * ROWS_PER_TILE, ROWS_PER_TILE)])


def _make_prop():
    mesh = plsc.VectorSubcoreMesh(core_axis_name="c", subcore_axis_name="s")
    return pl.kernel(
        _prop_body,
        mesh=mesh,
        out_type=jax.ShapeDtypeStruct((NC, NPAD, C), jnp.float32),
        scratch_types=[
            pltpu.VMEM((NCHUNK, CHUNK), jnp.int32),
            pltpu.VMEM((NCHUNK, CHUNK), jnp.int32),
            pltpu.VMEM((NCHUNK, CHUNK), jnp.float32),
            pltpu.VMEM((CHUNK, C), jnp.float32),
            pltpu.VMEM_SHARED((NPAD, C), jnp.float32),
        ],
        compiler_params=pltpu.CompilerParams(use_tc_tiling_on_sc=False),
    )


def kernel(features, edge_weight, edge_index, W1, b1, W2, b2):
    h0, ah0 = _mlp(features, W1, b1, W2, b2)
    src = edge_index[0].reshape(NC, NS, NCHUNK, CHUNK)
    dst = edge_index[1].reshape(NC, NS, NCHUNK, CHUNK)
    w = edge_weight.reshape(NC, NS, NCHUNK, CHUNK)
    pad = ((0, NPAD - N), (0, 0))
    ah0p = jnp.pad(ah0, pad)
    init = jnp.stack([ah0p, jnp.zeros_like(ah0p)])
    prop = _make_prop()
    h = jnp.pad(h0, pad)
    for _ in range(K):
        partials = prop(h, src, dst, w, init)
        h = _combine(partials)
    return h[:N]


# speedup vs baseline: 14.0773x; 1.7850x over previous
import functools

import jax
import jax.numpy as jnp
from jax import lax
from jax.experimental import pallas as pl
from jax.experimental.pallas import tpu as pltpu
from jax.experimental.pallas import tpu_sc as plsc

N = 10000
E = 320000
D = 128
H = 128
C = 64
K = 10
ALPHA = 0.1

NC = 2
NS = 16
LANES = 16
EDGES_PER_TILE = E // (NC * NS)
CHUNK = 80
NCHUNK = EDGES_PER_TILE // CHUNK
NPAD = 10240
ROWS_PER_TILE = NPAD // NS

ROW_BLK = 2000
CMB_BLK = 2048


def _mlp_body(x_ref, w1_ref, b1_ref, w2_ref, b2_ref, h_ref, ah_ref):
    h1 = jnp.maximum(
        jnp.dot(x_ref[...], w1_ref[...], preferred_element_type=jnp.float32)
        + b1_ref[...], 0.0)
    h2 = (jnp.dot(h1, w2_ref[...], preferred_element_type=jnp.float32)
          + b2_ref[...])
    h_ref[...] = h2
    ah_ref[...] = ALPHA * h2


def _mlp(features, W1, b1, W2, b2):
    grid = (N // ROW_BLK,)
    return pl.pallas_call(
        _mlp_body,
        grid=grid,
        in_specs=[
            pl.BlockSpec((ROW_BLK, D), lambda i: (i, 0)),
            pl.BlockSpec((D, H), lambda i: (0, 0)),
            pl.BlockSpec((1, H), lambda i: (0, 0)),
            pl.BlockSpec((H, C), lambda i: (0, 0)),
            pl.BlockSpec((1, C), lambda i: (0, 0)),
        ],
        out_specs=[
            pl.BlockSpec((ROW_BLK, C), lambda i: (i, 0)),
            pl.BlockSpec((ROW_BLK, C), lambda i: (i, 0)),
        ],
        out_shape=[
            jax.ShapeDtypeStruct((N, C), jnp.float32),
            jax.ShapeDtypeStruct((N, C), jnp.float32),
        ],
    )(features, W1, b1.reshape(1, H), W2, b2.reshape(1, C))


def _combine_body(p_ref, o_ref):
    o_ref[...] = p_ref[0] + p_ref[1]


def _combine(partials):
    grid = (NPAD // CMB_BLK,)
    return pl.pallas_call(
        _combine_body,
        grid=grid,
        in_specs=[pl.BlockSpec((NC, CMB_BLK, C), lambda i: (0, i, 0))],
        out_specs=pl.BlockSpec((CMB_BLK, C), lambda i: (i, 0)),
        out_shape=jax.ShapeDtypeStruct((NPAD, C), jnp.float32),
    )(partials)


_GATHER_DNUMS = lax.GatherDimensionNumbers(
    offset_dims=(), collapsed_slice_dims=(0,), start_index_map=(0,))


def _bcast_lane(vec, lane):
    idx = jnp.full((LANES, 1), lane, jnp.int32)
    return lax.gather(vec, idx, _GATHER_DNUMS, slice_sizes=(1,),
                      mode=lax.GatherScatterMode.PROMISE_IN_BOUNDS)


NBUF = 5


def _prop_body(h_hbm, src_hbm, dst_hbm, w_hbm, init_hbm, out_hbm,
               src_v, dst_v, w_v, rin_v, rout_v, agg_sh, gsem, ssem):
    c = lax.axis_index("c")
    s = lax.axis_index("s")

    pltpu.sync_copy(src_hbm.at[c, s], src_v)
    pltpu.sync_copy(dst_hbm.at[c, s], dst_v)
    pltpu.sync_copy(w_hbm.at[c, s], w_v)
    pltpu.sync_copy(init_hbm.at[c].at[pl.ds(s * ROWS_PER_TILE, ROWS_PER_TILE)],
                    agg_sh.at[pl.ds(s * ROWS_PER_TILE, ROWS_PER_TILE)])
    plsc.subcore_barrier()

    for b in range(NBUF):
        pltpu.async_copy(h_hbm.at[src_v.at[b]], rin_v.at[b], gsem.at[b])

    @pl.loop(0, NCHUNK, step=NBUF)
    def _(g0):
        for b in range(NBUF):
            j = g0 + b
            pltpu.make_async_copy(h_hbm.at[src_v.at[j]], rin_v.at[b],
                                  gsem.at[b]).wait()
            @pl.when(g0 > 0)
            def _():
                pltpu.make_async_copy(rout_v.at[b], agg_sh.at[dst_v.at[j]],
                                      ssem.at[b]).wait()
            for e5 in range(CHUNK // LANES):
                w16 = w_v[j, pl.ds(e5 * LANES, LANES)] * (1.0 - ALPHA)
                for e in range(LANES):
                    wb = _bcast_lane(w16, e)
                    row = e5 * LANES + e
                    for f in range(C // LANES):
                        sl = (row, pl.ds(f * LANES, LANES))
                        rout_v[(b,) + sl] = rin_v[(b,) + sl] * wb
            @pl.when(j + NBUF < NCHUNK)
            def _():
                pltpu.async_copy(h_hbm.at[src_v.at[j + NBUF]], rin_v.at[b],
                                 gsem.at[b])
            pltpu.async_copy(rout_v.at[b], agg_sh.at[dst_v.at[j]],
                             ssem.at[b], add=True)

    for b in range(NBUF):
        pltpu.make_async_copy(rout_v.at[b],
                              agg_sh.at[dst_v.at[NCHUNK - NBUF + b]],
                              ssem.at[b]).wait()

    plsc.subcore_barrier()
    pltpu.sync_copy(agg_sh.at[pl.ds(s * ROWS_PER_TILE, ROWS_PER_TILE)],
                    out_hbm.at[c].at[pl.ds(s * ROWS_PER_TILE, ROWS_PER_TILE)])


def _make_prop():
    mesh = plsc.VectorSubcoreMesh(core_axis_name="c", subcore_axis_name="s")
    return pl.kernel(
        _prop_body,
        mesh=mesh,
        out_type=jax.ShapeDtypeStruct((NC, NPAD, C), jnp.float32),
        scratch_types=[
            pltpu.VMEM((NCHUNK, CHUNK), jnp.int32),
            pltpu.VMEM((NCHUNK, CHUNK), jnp.int32),
            pltpu.VMEM((NCHUNK, CHUNK), jnp.float32),
            pltpu.VMEM((NBUF, CHUNK, C), jnp.float32),
            pltpu.VMEM((NBUF, CHUNK, C), jnp.float32),
            pltpu.VMEM_SHARED((NPAD, C), jnp.float32),
            pltpu.SemaphoreType.DMA((NBUF,)),
            pltpu.SemaphoreType.DMA((NBUF,)),
        ],
        compiler_params=pltpu.CompilerParams(use_tc_tiling_on_sc=False),
    )


def kernel(features, edge_weight, edge_index, W1, b1, W2, b2):
    h0, ah0 = _mlp(features, W1, b1, W2, b2)
    src = edge_index[0].reshape(NC, NS, NCHUNK, CHUNK)
    dst = edge_index[1].reshape(NC, NS, NCHUNK, CHUNK)
    w = edge_weight.reshape(NC, NS, NCHUNK, CHUNK)
    pad = ((0, NPAD - N), (0, 0))
    ah0p = jnp.pad(ah0, pad)
    init = jnp.stack([ah0p, jnp.zeros_like(ah0p)])
    prop = _make_prop()
    h = jnp.pad(h0, pad)
    for _ in range(K):
        partials = prop(h, src, dst, w, init)
        h = _combine(partials)
    return h[:N]
